# Initial kernel scaffold; baseline (speedup 1.0000x reference)
import jax
import jax.numpy as jnp
from jax.experimental import pallas as pl


def kernel(node_feat, edge_index, W1, W2, W3, W_cls):
    raise NotImplementedError("write your pallas kernel here")


# speedup vs baseline: 4.8390x; 4.8390x over previous
import functools

import jax
import jax.numpy as jnp
from jax import lax
from jax.experimental import pallas as pl
from jax.experimental.pallas import tpu as pltpu
from jax.experimental.pallas import tpu_sc as plsc

N = 10000
E = 320000
D = 128
H = 512
C = 2

NCORE = 2
NSUB = 16
NW = NCORE * NSUB

ROWS_MAIN = 624
ROWS_TAIL = N - 15 * ROWS_MAIN
RB = 208

R_BLK = 400
GRID = N // R_BLK


def _mesh():
    return plsc.VectorSubcoreMesh(core_axis_name="c", subcore_axis_name="s")


_HB = 400


def _hist_body(dst_hbm, out_hbm, idx_v, ones_v, buf_v, hist_sh, sem):
    c = lax.axis_index("c")
    s = lax.axis_index("s")
    for j in range(ROWS_TAIL // 16):
        buf_v[pl.ds(j * 16, 16)] = jnp.zeros((16,), jnp.float32)
    for j in range(_HB // 16):
        ones_v[pl.ds(j * 16, 16)] = jnp.ones((16,), jnp.float32)
    pltpu.sync_copy(buf_v.at[pl.ds(0, ROWS_MAIN)],
                    hist_sh.at[pl.ds(s * ROWS_MAIN, ROWS_MAIN)])

    @pl.when(s == 15)
    def _():
        pltpu.sync_copy(buf_v.at[pl.ds(0, ROWS_TAIL - ROWS_MAIN)],
                        hist_sh.at[pl.ds(15 * ROWS_MAIN, ROWS_TAIL - ROWS_MAIN)])

    plsc.subcore_barrier()
    base0 = (c * NSUB + s) * (E // NW)

    def body(i, carry):
        base = base0 + i * _HB
        pltpu.sync_copy(dst_hbm.at[pl.ds(base, _HB)], idx_v)
        pltpu.sync_copy(ones_v, hist_sh.at[idx_v], add=True)
        return carry

    lax.fori_loop(0, E // NW // _HB, body, 0)
    plsc.subcore_barrier()
    nr = jnp.where(s == 15, ROWS_TAIL, ROWS_MAIN)
    del nr
    pltpu.sync_copy(hist_sh.at[pl.ds(s * ROWS_MAIN, ROWS_MAIN)],
                    buf_v.at[pl.ds(0, ROWS_MAIN)])
    pltpu.sync_copy(buf_v.at[pl.ds(0, ROWS_MAIN)],
                    out_hbm.at[pl.ds(c * N + s * ROWS_MAIN, ROWS_MAIN)])

    @pl.when(s == 15)
    def _():
        tail0 = 15 * ROWS_MAIN + ROWS_MAIN
        tn = ROWS_TAIL - ROWS_MAIN
        pltpu.sync_copy(hist_sh.at[pl.ds(tail0, tn)], buf_v.at[pl.ds(0, tn)])
        pltpu.sync_copy(buf_v.at[pl.ds(0, tn)],
                        out_hbm.at[pl.ds(c * N + tail0, tn)])


def _hist_call(dst):
    return pl.kernel(
        _hist_body,
        out_type=jax.ShapeDtypeStruct((NCORE * N,), jnp.float32),
        mesh=_mesh(),
        scratch_types=[
            pltpu.VMEM((_HB,), jnp.int32),
            pltpu.VMEM((_HB,), jnp.float32),
            pltpu.VMEM((ROWS_TAIL,), jnp.float32),
            pltpu.VMEM_SHARED((N,), jnp.float32),
            pltpu.SemaphoreType.DMA,
        ],
    )(dst)


_SB = 200


def _make_spmm(nc, fc):
    nb = E // NW // _SB

    def body(*refs):
        xps = refs[0:nc]
        src = refs[nc]
        dst = refs[nc + 1]
        outs = refs[nc + 2:2 * nc + 2]
        sidx, didx, rows, acc_sh, sem = refs[2 * nc + 2:]
        c = lax.axis_index("c")
        s = lax.axis_index("s")
        base0 = (c * NSUB + s) * (E // NW)
        steps = [(0, _SB), (_SB, _SB), (2 * _SB, _SB), (3 * _SB, 24)]

        def bounce(src_ref, dst_ref, r0_src, r0_dst):
            for off, sz in steps:
                pltpu.sync_copy(src_ref.at[pl.ds(r0_src + off, sz)],
                                rows.at[pl.ds(0, sz)])
                pltpu.sync_copy(rows.at[pl.ds(0, sz)],
                                dst_ref.at[pl.ds(r0_dst + off, sz)])

        for ci in range(nc):
            bounce(xps[ci], acc_sh, s * ROWS_MAIN, s * ROWS_MAIN)

            @pl.when(s == 15)
            def _():
                t0 = N - 16
                pltpu.sync_copy(xps[ci].at[pl.ds(t0, 16)], rows.at[pl.ds(0, 16)])
                pltpu.sync_copy(rows.at[pl.ds(0, 16)], acc_sh.at[pl.ds(t0, 16)])

            plsc.subcore_barrier()

            def ebody(i, carry):
                base = base0 + i * _SB
                pltpu.sync_copy(src.at[pl.ds(base, _SB)], sidx)
                pltpu.sync_copy(dst.at[pl.ds(base, _SB)], didx)
                pltpu.async_copy(xps[ci].at[sidx], rows, sem).wait()
                pltpu.sync_copy(rows, acc_sh.at[didx], add=True)
                return carry

            lax.fori_loop(0, nb, ebody, 0)
            plsc.subcore_barrier()
            bounce(acc_sh, outs[ci], s * ROWS_MAIN, c * N + s * ROWS_MAIN)

            @pl.when(s == 15)
            def _():
                t0 = N - 16
                pltpu.sync_copy(acc_sh.at[pl.ds(t0, 16)], rows.at[pl.ds(0, 16)])
                pltpu.sync_copy(rows.at[pl.ds(0, 16)],
                                outs[ci].at[pl.ds(c * N + t0, 16)])

            plsc.subcore_barrier()

    def call(xps, src, dst):
        return pl.kernel(
            body,
            out_type=[jax.ShapeDtypeStruct((NCORE * N, fc), jnp.float32)] * nc,
            mesh=_mesh(),
            scratch_types=[
                pltpu.VMEM((_SB,), jnp.int32),
                pltpu.VMEM((_SB,), jnp.int32),
                pltpu.VMEM((_SB, fc), jnp.float32),
                pltpu.VMEM_SHARED((N, fc), jnp.float32),
                pltpu.SemaphoreType.DMA,
            ],
        )(*xps, src, dst)

    return call


_EB = 80


def _edge_body(h_hbm, src, dst, wcls, out_hbm,
               sidx, didx, rows_s, rows_d, wc_v, lg_v, sem_s, sem_d):
    c = lax.axis_index("c")
    s = lax.axis_index("s")
    w = c * NSUB + s
    ept = E // NW
    pltpu.sync_copy(wcls, wc_v)
    iota = lax.iota(jnp.int32, 16)

    def body(i, carry):
        base = w * ept + i * _EB
        pltpu.sync_copy(src.at[pl.ds(base, _EB)], sidx)
        pltpu.sync_copy(dst.at[pl.ds(base, _EB)], didx)
        cp_s = pltpu.async_copy(h_hbm.at[sidx], rows_s, sem_s)
        cp_d = pltpu.async_copy(h_hbm.at[didx], rows_d, sem_d)
        cp_s.wait()
        cp_d.wait()

        def edge(e, carry2):
            acc0 = jnp.zeros((16,), jnp.float32)
            acc1 = jnp.zeros((16,), jnp.float32)
            for j in range(H // 16):
                hs = rows_s[e, pl.ds(j * 16, 16)]
                hd = rows_d[e, pl.ds(j * 16, 16)]
                p = hs * hd
                acc0 = acc0 + p * wc_v[0, pl.ds(j * 16, 16)]
                acc1 = acc1 + p * wc_v[1, pl.ds(j * 16, 16)]
            lg_v[pl.ds(32 * e, 16)] = acc0
            lg_v[pl.ds(32 * e + 16, 16)] = acc1
            return carry2

        lax.fori_loop(0, _EB, edge, 0)
        pltpu.sync_copy(lg_v,
                        out_hbm.at[pl.ds(32 * (w * ept + i * _EB), 32 * _EB)])
        return carry

    lax.fori_loop(0, ept // _EB, body, 0)


def _edge_call(h, src, dst, wcls):
    return pl.kernel(
        _edge_body,
        out_type=jax.ShapeDtypeStruct((2 * E * 16,), jnp.float32),
        mesh=_mesh(),
        scratch_types=[
            pltpu.VMEM((_EB,), jnp.int32),
            pltpu.VMEM((_EB,), jnp.int32),
            pltpu.VMEM((_EB, H), jnp.float32),
            pltpu.VMEM((_EB, H), jnp.float32),
            pltpu.VMEM((C, H), jnp.float32),
            pltpu.VMEM((32 * _EB,), jnp.float32),
            pltpu.SemaphoreType.DMA,
            pltpu.SemaphoreType.DMA,
        ],
    )(h, src, dst, wcls)


_RED_BLK = 4000


def _reduce_body(p_ref, o_ref):
    o_ref[...] = jnp.sum(p_ref[...], axis=1, keepdims=True)


def _reduce_call(partials):
    return pl.pallas_call(
        _reduce_body,
        grid=(2 * E // _RED_BLK,),
        in_specs=[pl.BlockSpec((_RED_BLK, 16), lambda i: (i, 0))],
        out_specs=pl.BlockSpec((_RED_BLK, 1), lambda i: (i, 0)),
        out_shape=jax.ShapeDtypeStruct((2 * E, 1), jnp.float32),
    )(partials)


def _prep_body(h0_ref, h1_ref, nf_ref, dis_ref, xp0_ref):
    deg = h0_ref[...] + h1_ref[...] + 1.0
    dis = lax.rsqrt(deg)
    dis_ref[...] = dis
    xp0_ref[...] = nf_ref[...] * dis


def _prep_call(hist, node_feat):
    h0 = hist[0:N].reshape(N, 1)
    h1 = hist[N:2 * N].reshape(N, 1)
    return pl.pallas_call(
        _prep_body,
        grid=(GRID,),
        in_specs=[
            pl.BlockSpec((R_BLK, 1), lambda i: (i, 0)),
            pl.BlockSpec((R_BLK, 1), lambda i: (i, 0)),
            pl.BlockSpec((R_BLK, D), lambda i: (i, 0)),
        ],
        out_specs=[
            pl.BlockSpec((R_BLK, 1), lambda i: (i, 0)),
            pl.BlockSpec((R_BLK, D), lambda i: (i, 0)),
        ],
        out_shape=[
            jax.ShapeDtypeStruct((N, 1), jnp.float32),
            jax.ShapeDtypeStruct((N, D), jnp.float32),
        ],
    )(h0, h1, node_feat)


def _make_layer(nc, fc, last):
    din = nc * fc
    nblk = N // R_BLK

    def body(*refs):
        acc_a = refs[0:nc]
        acc_b = refs[nc:2 * nc]
        xps = refs[2 * nc:3 * nc]
        dis_ref = refs[3 * nc]
        w_ref = refs[3 * nc + 1]
        outs = refs[3 * nc + 2:]
        dis = dis_ref[...]
        parts = []
        for ci in range(nc):
            parts.append((acc_a[ci][...] + acc_b[ci][...] - xps[ci][...]) * dis)
        a = jnp.concatenate(parts, axis=1) if nc > 1 else parts[0]
        z = lax.dot_general(a, w_ref[...], (((1,), (1,)), ((), ())),
                            preferred_element_type=jnp.float32)
        hv = jnp.maximum(z, 0.0)
        if last:
            outs[0][...] = hv
        else:
            hd = hv * dis
            for k in range(H // 128):
                outs[k][...] = hd[:, k * 128:(k + 1) * 128]

    def call(accs, xps, dis, w):
        in_specs = (
            [pl.BlockSpec((R_BLK, fc), lambda i: (i, 0)) for _ in range(nc)]
            + [pl.BlockSpec((R_BLK, fc), lambda i: (nblk + i, 0))
               for _ in range(nc)]
            + [pl.BlockSpec((R_BLK, fc), lambda i: (i, 0)) for _ in range(nc)]
            + [pl.BlockSpec((R_BLK, 1), lambda i: (i, 0)),
               pl.BlockSpec((H, din), lambda i: (0, 0))]
        )
        if last:
            out_specs = [pl.BlockSpec((R_BLK, H), lambda i: (i, 0))]
            out_shape = [jax.ShapeDtypeStruct((N, H), jnp.float32)]
        else:
            out_specs = [pl.BlockSpec((R_BLK, 128), lambda i: (i, 0))
                         for _ in range(H // 128)]
            out_shape = [jax.ShapeDtypeStruct((N, 128), jnp.float32)
                         for _ in range(H // 128)]
        return pl.pallas_call(
            body,
            grid=(GRID,),
            in_specs=in_specs,
            out_specs=out_specs,
            out_shape=out_shape,
        )(*accs, *accs, *xps, dis, w)

    return call


_spmm1 = _make_spmm(1, 128)
_spmm128 = _make_spmm(4, 128)
_layer1 = _make_layer(1, 128, last=False)
_layer2 = _make_layer(4, 128, last=False)
_layer3 = _make_layer(4, 128, last=True)


def kernel(node_feat, edge_index, W1, W2, W3, W_cls):
    src = edge_index[0]
    dst = edge_index[1]

    hist = _hist_call(dst)
    dis, xp1 = _prep_call(hist, node_feat)

    acc1 = _spmm1([xp1], src, dst)
    xp2 = _layer1(acc1, [xp1], dis, W1)

    acc2 = _spmm128(xp2, src, dst)
    xp3 = _layer2(acc2, xp2, dis, W2)

    acc3 = _spmm128(xp3, src, dst)
    (h,) = _layer3(acc3, xp3, dis, W3)

    partials = _edge_call(h, src, dst, W_cls)
    logits = _reduce_call(partials.reshape(2 * E, 16))
    return (logits.reshape(E, 2), h)
